# baseline (speedup 1.0000x reference)
import functools

import jax
import jax.numpy as jnp
from jax.experimental import pallas as pl
from jax.experimental.pallas import tpu as pltpu


def _fold_sum(x, num_neigh):
    half = num_neigh // 2
    if num_neigh % 2 == 0 and half % 8 == 0:
        return jnp.sum(x[:, :half, :] + x[:, half:, :], axis=1)
    return jnp.sum(x, axis=1)


def _mean_kernel(a_ref, b_ref, o_ref, *, num_neigh, half_tile, inv_n):
    sa = _fold_sum(a_ref[...].astype(jnp.float32), num_neigh)
    sb = _fold_sum(b_ref[...].astype(jnp.float32), num_neigh)
    o_ref[pl.ds(0, half_tile), :] = (sa * inv_n).astype(o_ref.dtype)
    o_ref[pl.ds(half_tile, half_tile), :] = (sb * inv_n).astype(o_ref.dtype)


def kernel(neighbor_feature):
    num_src, num_neigh, input_dim = neighbor_feature.shape
    dtype = neighbor_feature.dtype
    itemsize = jnp.dtype(dtype).itemsize

    row_bytes = num_neigh * input_dim * itemsize
    half_tile = max(16, min(num_src // 2, (8 << 20) // max(row_bytes, 1)))
    half_tile -= half_tile % 16
    while half_tile > 16 and num_src % (2 * half_tile) != 0:
        half_tile -= 16
    tile = 2 * half_tile
    grid = (num_src // tile,)

    kfn = functools.partial(_mean_kernel, num_neigh=num_neigh,
                            half_tile=half_tile, inv_n=1.0 / float(num_neigh))

    in_bytes = tile * row_bytes
    out_bytes = tile * input_dim * itemsize
    vmem_limit = int(min(100 << 20, 2 * in_bytes + 2 * out_bytes + (4 << 20)))

    in_block = (half_tile, num_neigh, input_dim)
    return pl.pallas_call(
        kfn,
        out_shape=jax.ShapeDtypeStruct((num_src, input_dim), dtype),
        grid=grid,
        in_specs=[
            pl.BlockSpec(in_block, lambda i: (2 * i, 0, 0)),
            pl.BlockSpec(in_block, lambda i: (2 * i + 1, 0, 0)),
        ],
        out_specs=pl.BlockSpec((tile, input_dim), lambda i: (i, 0)),
        compiler_params=pltpu.CompilerParams(
            dimension_semantics=("parallel",),
            vmem_limit_bytes=vmem_limit,
        ),
        cost_estimate=pl.CostEstimate(
            flops=num_src * num_neigh * input_dim,
            transcendentals=0,
            bytes_accessed=num_src * (num_neigh + 1) * input_dim * itemsize,
        ),
    )(neighbor_feature, neighbor_feature)


# speedup vs baseline: 1.0141x; 1.0141x over previous
import functools

import jax
import jax.numpy as jnp
from jax.experimental import pallas as pl
from jax.experimental.pallas import tpu as pltpu


def _mean_kernel(x_ref, o_ref, *, num_neigh, inv_n):
    x = x_ref[...].astype(jnp.float32)
    half = num_neigh // 2
    if num_neigh % 2 == 0 and half % 8 == 0:
        s = jnp.sum(x[:, :half, :] + x[:, half:, :], axis=1)
    else:
        s = jnp.sum(x, axis=1)
    o_ref[...] = (s * inv_n).astype(o_ref.dtype)


def _pick_tile(num_src, row_bytes):
    tile = max(8, min(num_src, (8 << 20) // max(row_bytes, 1)))
    tile -= tile % 8
    while tile > 8 and num_src % tile != 0:
        tile -= 8
    return tile


def kernel(neighbor_feature):
    num_src, num_neigh, input_dim = neighbor_feature.shape
    dtype = neighbor_feature.dtype
    itemsize = jnp.dtype(dtype).itemsize

    row_bytes = num_neigh * input_dim * itemsize
    tile = _pick_tile(num_src, row_bytes)
    grid = (pl.cdiv(num_src, tile),)

    kfn = functools.partial(_mean_kernel, num_neigh=num_neigh,
                            inv_n=1.0 / float(num_neigh))

    in_bytes = tile * row_bytes
    out_bytes = tile * input_dim * itemsize
    vmem_limit = int(min(100 << 20, 2 * in_bytes + 2 * out_bytes + (4 << 20)))

    return pl.pallas_call(
        kfn,
        out_shape=jax.ShapeDtypeStruct((num_src, input_dim), dtype),
        grid=grid,
        in_specs=[pl.BlockSpec((tile, num_neigh, input_dim),
                               lambda i: (i, 0, 0))],
        out_specs=pl.BlockSpec((tile, input_dim), lambda i: (i, 0)),
        compiler_params=pltpu.CompilerParams(
            dimension_semantics=("parallel",),
            vmem_limit_bytes=vmem_limit,
        ),
        cost_estimate=pl.CostEstimate(
            flops=num_src * num_neigh * input_dim,
            transcendentals=0,
            bytes_accessed=num_src * (num_neigh + 1) * input_dim * itemsize,
        ),
    )(neighbor_feature)
